# baseline (device time: 92032 ns/iter reference)
import jax
import jax.numpy as jnp
from jax import lax
from jax.experimental import pallas as pl
from jax.experimental.pallas import tpu as pltpu

N_DEV = 4
N_PEER = N_DEV - 1
UNROLL = 8


def _fused(table, local_idx, mask):
    v_per, d = table.shape
    n = local_idx.shape[0]
    chunk = n // N_DEV

    def body(table_ref, loc_ref, mask_ref, out_ref,
             gat_ref, gsems, sbuf_ref, ra_ref, rb_ref,
             sa_sems, ra_sems, sb_sems, rb_sems):
        my = lax.axis_index("i")

        barrier_sem = pltpu.get_barrier_semaphore()
        for j in range(N_PEER):
            pl.semaphore_signal(
                barrier_sem, inc=1,
                device_id=((my + j + 1) % N_DEV,),
                device_id_type=pl.DeviceIdType.MESH,
            )
        pl.semaphore_wait(barrier_sem, N_PEER)

        def issue_chunk(c, sem):
            base = c * chunk

            def issue(k, cnt):
                for u in range(UNROLL):
                    i = base + k * UNROLL + u
                    r = loc_ref[i]
                    owned = (r >= 0) & (r < v_per)

                    @pl.when(owned)
                    def _():
                        pltpu.make_async_copy(
                            table_ref.at[pl.ds(r, 1), :],
                            gat_ref.at[pl.ds(i, 1), :],
                            sem,
                        ).start()

                    cnt = cnt + owned.astype(jnp.int32)
                return cnt

            return lax.fori_loop(0, chunk // UNROLL, issue, jnp.int32(0))

        def drain_chunk(cnt, sem):

            def drain(k, carry):
                pltpu.make_async_copy(
                    table_ref.at[pl.ds(0, 1), :],
                    gat_ref.at[pl.ds(0, 1), :],
                    sem,
                ).wait()
                return carry

            lax.fori_loop(0, cnt, drain, 0)

        def masked_chunk(c):
            base = c * chunk
            return jnp.where(
                mask_ref[pl.ds(base, chunk), :] != 0.0,
                gat_ref[pl.ds(base, chunk), :],
                0.0,
            )

        cnts = []
        for j in range(N_PEER):
            cnts.append(issue_chunk((my + j + 1) % N_DEV, gsems.at[j]))
        cnt_own = issue_chunk(my, gsems.at[N_PEER])

        p1 = []
        for j in range(N_PEER):
            dst = (my + j + 1) % N_DEV
            c = dst
            drain_chunk(cnts[j], gsems.at[j])
            sbuf_ref[j, :, :] = masked_chunk(c).astype(sbuf_ref.dtype)
            rdma = pltpu.make_async_remote_copy(
                src_ref=sbuf_ref.at[j],
                dst_ref=ra_ref.at[j],
                send_sem=sa_sems.at[j],
                recv_sem=ra_sems.at[j],
                device_id=(dst,),
                device_id_type=pl.DeviceIdType.MESH,
            )
            rdma.start()
            p1.append(rdma)

        drain_chunk(cnt_own, gsems.at[N_PEER])
        mybase = my * chunk
        out_ref[pl.ds(mybase, chunk), :] = masked_chunk(my).astype(out_ref.dtype)

        for rdma in p1:
            rdma.wait_recv()
        out_ref[pl.ds(mybase, chunk), :] = (
            out_ref[pl.ds(mybase, chunk), :]
            + ra_ref[0, :, :] + ra_ref[1, :, :] + ra_ref[2, :, :]
        )

        p2 = []
        for j in range(N_PEER):
            dst = (my + j + 1) % N_DEV
            rdma = pltpu.make_async_remote_copy(
                src_ref=out_ref.at[pl.ds(mybase, chunk), :],
                dst_ref=rb_ref.at[j],
                send_sem=sb_sems.at[j],
                recv_sem=rb_sems.at[j],
                device_id=(dst,),
                device_id_type=pl.DeviceIdType.MESH,
            )
            rdma.start()
            p2.append(rdma)

        for j in range(N_PEER):
            p2[j].wait_recv()
            src = (my - 1 - j) % N_DEV
            out_ref[pl.ds(src * chunk, chunk), :] = rb_ref[j, :, :]

        for rdma in p1:
            rdma.wait_send()
        for rdma in p2:
            rdma.wait_send()

    return pl.pallas_call(
        body,
        out_shape=jax.ShapeDtypeStruct((n, d), jnp.bfloat16),
        in_specs=[
            pl.BlockSpec(memory_space=pl.ANY),
            pl.BlockSpec(memory_space=pltpu.SMEM),
            pl.BlockSpec(memory_space=pltpu.VMEM),
        ],
        out_specs=pl.BlockSpec(memory_space=pltpu.VMEM),
        scratch_shapes=[
            pltpu.VMEM((n, d), jnp.float32),
            pltpu.SemaphoreType.DMA((N_DEV,)),
            pltpu.VMEM((N_PEER, chunk, d), jnp.bfloat16),
            pltpu.VMEM((N_PEER, chunk, d), jnp.bfloat16),
            pltpu.VMEM((N_PEER, chunk, d), jnp.bfloat16),
            pltpu.SemaphoreType.DMA((N_PEER,)),
            pltpu.SemaphoreType.DMA((N_PEER,)),
            pltpu.SemaphoreType.DMA((N_PEER,)),
            pltpu.SemaphoreType.DMA((N_PEER,)),
        ],
        compiler_params=pltpu.CompilerParams(collective_id=0),
    )(table, local_idx, mask)


def kernel(table, idx):
    v_per = table.shape[0]
    my = lax.axis_index("i")
    local = idx.astype(jnp.int32) - my * v_per
    mask = (local >= 0) & (local < v_per)
    maskf = mask.astype(jnp.float32)[:, None]
    return _fused(table, local, maskf)


# device time: 88499 ns/iter; 1.0399x vs baseline; 1.0399x over previous
import jax
import jax.numpy as jnp
from jax import lax
from jax.experimental import pallas as pl
from jax.experimental.pallas import tpu as pltpu

N_DEV = 4
N_PEER = N_DEV - 1
UNROLL = 16


def _fused(table, local_idx, mask):
    v_per, d = table.shape
    n = local_idx.shape[0]
    chunk = n // N_DEV

    def body(table_ref, loc_ref, mask_ref, out_ref,
             gat_ref, gsems, sbuf_ref, ra_ref, rb_ref,
             sa_sems, ra_sems, sb_sems, rb_sems):
        my = lax.axis_index("i")

        barrier_sem = pltpu.get_barrier_semaphore()
        for j in range(N_PEER):
            pl.semaphore_signal(
                barrier_sem, inc=1,
                device_id=((my + j + 1) % N_DEV,),
                device_id_type=pl.DeviceIdType.MESH,
            )
        pl.semaphore_wait(barrier_sem, N_PEER)

        def issue_chunk(c, sem):
            base = c * chunk

            def issue(k, cnt):
                for u in range(UNROLL):
                    i = base + k * UNROLL + u
                    r = loc_ref[i]
                    owned = r >= 0

                    @pl.when(owned)
                    def _():
                        pltpu.make_async_copy(
                            table_ref.at[pl.ds(r, 1), :],
                            gat_ref.at[pl.ds(i, 1), :],
                            sem,
                        ).start()

                    cnt = cnt + owned.astype(jnp.int32)
                return cnt

            return lax.fori_loop(0, chunk // UNROLL, issue, jnp.int32(0))

        def drain_chunk(cnt, sem):

            def drain(k, carry):
                pltpu.make_async_copy(
                    table_ref.at[pl.ds(0, 1), :],
                    gat_ref.at[pl.ds(0, 1), :],
                    sem,
                ).wait()
                return carry

            lax.fori_loop(0, cnt, drain, 0)

        def masked_chunk(c):
            base = c * chunk
            return jnp.where(
                mask_ref[pl.ds(base, chunk), :] != 0.0,
                gat_ref[pl.ds(base, chunk), :],
                0.0,
            )

        cnts = []
        for j in range(N_PEER):
            cnts.append(issue_chunk((my + j + 1) % N_DEV, gsems.at[j]))
        cnt_own = issue_chunk(my, gsems.at[N_PEER])

        p1 = []
        for j in range(N_PEER):
            dst = (my + j + 1) % N_DEV
            c = dst
            drain_chunk(cnts[j], gsems.at[j])
            sbuf_ref[j, :, :] = masked_chunk(c).astype(sbuf_ref.dtype)
            rdma = pltpu.make_async_remote_copy(
                src_ref=sbuf_ref.at[j],
                dst_ref=ra_ref.at[j],
                send_sem=sa_sems.at[j],
                recv_sem=ra_sems.at[j],
                device_id=(dst,),
                device_id_type=pl.DeviceIdType.MESH,
            )
            rdma.start()
            p1.append(rdma)

        drain_chunk(cnt_own, gsems.at[N_PEER])
        mybase = my * chunk
        out_ref[pl.ds(mybase, chunk), :] = masked_chunk(my).astype(out_ref.dtype)

        for rdma in p1:
            rdma.wait_recv()
        out_ref[pl.ds(mybase, chunk), :] = (
            out_ref[pl.ds(mybase, chunk), :]
            + ra_ref[0, :, :] + ra_ref[1, :, :] + ra_ref[2, :, :]
        )

        p2 = []
        for j in range(N_PEER):
            dst = (my + j + 1) % N_DEV
            rdma = pltpu.make_async_remote_copy(
                src_ref=out_ref.at[pl.ds(mybase, chunk), :],
                dst_ref=rb_ref.at[j],
                send_sem=sb_sems.at[j],
                recv_sem=rb_sems.at[j],
                device_id=(dst,),
                device_id_type=pl.DeviceIdType.MESH,
            )
            rdma.start()
            p2.append(rdma)

        for j in range(N_PEER):
            p2[j].wait_recv()
            src = (my - 1 - j) % N_DEV
            out_ref[pl.ds(src * chunk, chunk), :] = rb_ref[j, :, :]

        for rdma in p1:
            rdma.wait_send()
        for rdma in p2:
            rdma.wait_send()

    return pl.pallas_call(
        body,
        out_shape=jax.ShapeDtypeStruct((n, d), jnp.bfloat16),
        in_specs=[
            pl.BlockSpec(memory_space=pl.ANY),
            pl.BlockSpec(memory_space=pltpu.SMEM),
            pl.BlockSpec(memory_space=pltpu.VMEM),
        ],
        out_specs=pl.BlockSpec(memory_space=pltpu.VMEM),
        scratch_shapes=[
            pltpu.VMEM((n, d), jnp.float32),
            pltpu.SemaphoreType.DMA((N_DEV,)),
            pltpu.VMEM((N_PEER, chunk, d), jnp.bfloat16),
            pltpu.VMEM((N_PEER, chunk, d), jnp.bfloat16),
            pltpu.VMEM((N_PEER, chunk, d), jnp.bfloat16),
            pltpu.SemaphoreType.DMA((N_PEER,)),
            pltpu.SemaphoreType.DMA((N_PEER,)),
            pltpu.SemaphoreType.DMA((N_PEER,)),
            pltpu.SemaphoreType.DMA((N_PEER,)),
        ],
        compiler_params=pltpu.CompilerParams(collective_id=0),
    )(table, local_idx, mask)


def kernel(table, idx):
    v_per = table.shape[0]
    my = lax.axis_index("i")
    local = idx.astype(jnp.int32) - my * v_per
    mask = (local >= 0) & (local < v_per)
    local_enc = jnp.where(mask, local, -1)
    maskf = mask.astype(jnp.float32)[:, None]
    return _fused(table, local_enc, maskf)


# device time: 75002 ns/iter; 1.2271x vs baseline; 1.1800x over previous
import jax
import jax.numpy as jnp
from jax import lax
from jax.experimental import pallas as pl
from jax.experimental.pallas import tpu as pltpu

N_DEV = 4
N_PEER = N_DEV - 1
UNROLL = 16


def _fused(table, gather_lists, mask):
    v_per, d = table.shape
    n = mask.shape[0]
    chunk = n // N_DEV

    def body(table_ref, pack_ref, bound_ref, mask_ref, out_ref,
             gat_ref, gsems, sbuf_ref, ra_ref, rb_ref,
             sa_sems, ra_sems, sb_sems, rb_sems):
        my = lax.axis_index("i")

        barrier_sem = pltpu.get_barrier_semaphore()
        for j in range(N_PEER):
            pl.semaphore_signal(
                barrier_sem, inc=1,
                device_id=((my + j + 1) % N_DEV,),
                device_id_type=pl.DeviceIdType.MESH,
            )
        pl.semaphore_wait(barrier_sem, N_PEER)

        def issue_chunk(c, sem):

            def issue(k, carry):
                p = pack_ref[k]
                r = lax.shift_right_logical(p, 11)
                i = lax.bitwise_and(p, n - 1)
                pltpu.make_async_copy(
                    table_ref.at[pl.ds(r, 1), :],
                    gat_ref.at[pl.ds(i, 1), :],
                    sem,
                ).start()
                return carry

            lax.fori_loop(bound_ref[c], bound_ref[c + 1], issue, 0)
            return bound_ref[c + 1] - bound_ref[c]

        def drain_chunk(cnt, sem):

            def drain(k, carry):
                pltpu.make_async_copy(
                    table_ref.at[pl.ds(0, 1), :],
                    gat_ref.at[pl.ds(0, 1), :],
                    sem,
                ).wait()
                return carry

            lax.fori_loop(0, cnt, drain, 0)

        def masked_chunk(c):
            base = c * chunk
            return jnp.where(
                mask_ref[pl.ds(base, chunk), :] != 0.0,
                gat_ref[pl.ds(base, chunk), :],
                0.0,
            )

        cnts = []
        for j in range(N_PEER):
            cnts.append(issue_chunk((my + j + 1) % N_DEV, gsems.at[j]))
        cnt_own = issue_chunk(my, gsems.at[N_PEER])

        p1 = []
        for j in range(N_PEER):
            dst = (my + j + 1) % N_DEV
            c = dst
            drain_chunk(cnts[j], gsems.at[j])
            sbuf_ref[j, :, :] = masked_chunk(c).astype(sbuf_ref.dtype)
            rdma = pltpu.make_async_remote_copy(
                src_ref=sbuf_ref.at[j],
                dst_ref=ra_ref.at[j],
                send_sem=sa_sems.at[j],
                recv_sem=ra_sems.at[j],
                device_id=(dst,),
                device_id_type=pl.DeviceIdType.MESH,
            )
            rdma.start()
            p1.append(rdma)

        drain_chunk(cnt_own, gsems.at[N_PEER])
        mybase = my * chunk
        out_ref[pl.ds(mybase, chunk), :] = masked_chunk(my).astype(out_ref.dtype)

        for rdma in p1:
            rdma.wait_recv()
        out_ref[pl.ds(mybase, chunk), :] = (
            out_ref[pl.ds(mybase, chunk), :]
            + ra_ref[0, :, :] + ra_ref[1, :, :] + ra_ref[2, :, :]
        )

        p2 = []
        for j in range(N_PEER):
            dst = (my + j + 1) % N_DEV
            rdma = pltpu.make_async_remote_copy(
                src_ref=out_ref.at[pl.ds(mybase, chunk), :],
                dst_ref=rb_ref.at[j],
                send_sem=sb_sems.at[j],
                recv_sem=rb_sems.at[j],
                device_id=(dst,),
                device_id_type=pl.DeviceIdType.MESH,
            )
            rdma.start()
            p2.append(rdma)

        for j in range(N_PEER):
            p2[j].wait_recv()
            src = (my - 1 - j) % N_DEV
            out_ref[pl.ds(src * chunk, chunk), :] = rb_ref[j, :, :]

        for rdma in p1:
            rdma.wait_send()
        for rdma in p2:
            rdma.wait_send()

    return pl.pallas_call(
        body,
        out_shape=jax.ShapeDtypeStruct((n, d), jnp.bfloat16),
        in_specs=[
            pl.BlockSpec(memory_space=pl.ANY),
            pl.BlockSpec(memory_space=pltpu.SMEM),
            pl.BlockSpec(memory_space=pltpu.SMEM),
            pl.BlockSpec(memory_space=pltpu.VMEM),
        ],
        out_specs=pl.BlockSpec(memory_space=pltpu.VMEM),
        scratch_shapes=[
            pltpu.VMEM((n, d), jnp.float32),
            pltpu.SemaphoreType.DMA((N_DEV,)),
            pltpu.VMEM((N_PEER, chunk, d), jnp.bfloat16),
            pltpu.VMEM((N_PEER, chunk, d), jnp.bfloat16),
            pltpu.VMEM((N_PEER, chunk, d), jnp.bfloat16),
            pltpu.SemaphoreType.DMA((N_PEER,)),
            pltpu.SemaphoreType.DMA((N_PEER,)),
            pltpu.SemaphoreType.DMA((N_PEER,)),
            pltpu.SemaphoreType.DMA((N_PEER,)),
        ],
        compiler_params=pltpu.CompilerParams(collective_id=0),
    )(table, *gather_lists, mask)


def kernel(table, idx):
    v_per = table.shape[0]
    n = idx.shape[0]
    chunk = n // N_DEV
    my = lax.axis_index("i")
    local = idx.astype(jnp.int32) - my * v_per
    mask = (local >= 0) & (local < v_per)

    pos = jnp.arange(n, dtype=jnp.int32)
    pack = jnp.where(mask, jnp.left_shift(local, 11) | pos, 0)
    key = (~mask).astype(jnp.int32)
    _, packs = jax.lax.sort((key, pack), num_keys=1)
    cnts = jnp.sum(mask.reshape(N_DEV, chunk), axis=1).astype(jnp.int32)
    bounds = jnp.concatenate(
        [jnp.zeros((1,), jnp.int32), jnp.cumsum(cnts).astype(jnp.int32)]
    )

    maskf = mask.astype(jnp.float32)[:, None]
    return _fused(table, (packs, bounds), maskf)


# device time: 74456 ns/iter; 1.2361x vs baseline; 1.0073x over previous
import jax
import jax.numpy as jnp
from jax import lax
from jax.experimental import pallas as pl
from jax.experimental.pallas import tpu as pltpu

N_DEV = 4
N_PEER = N_DEV - 1
OFFS = (2, 1, 3)


def _fused(table, gather_lists, mask):
    v_per, d = table.shape
    n = mask.shape[0]
    chunk = n // N_DEV

    def body(table_ref, pack_ref, bound_ref, mask_ref, out_ref,
             gat_ref, gsems, sbuf_ref, ra_ref, rb_ref,
             sa_sems, ra_sems, sb_sems, rb_sems):
        my = lax.axis_index("i")

        barrier_sem = pltpu.get_barrier_semaphore()
        for j in range(N_PEER):
            pl.semaphore_signal(
                barrier_sem, inc=1,
                device_id=((my + j + 1) % N_DEV,),
                device_id_type=pl.DeviceIdType.MESH,
            )
        pl.semaphore_wait(barrier_sem, N_PEER)

        def issue_chunk(c, sem):

            def issue(k, carry):
                p = pack_ref[k]
                r = lax.shift_right_logical(p, 11)
                i = lax.bitwise_and(p, n - 1)
                pltpu.make_async_copy(
                    table_ref.at[pl.ds(r, 1), :],
                    gat_ref.at[pl.ds(i, 1), :],
                    sem,
                ).start()
                return carry

            lax.fori_loop(bound_ref[c], bound_ref[c + 1], issue, 0)
            return bound_ref[c + 1] - bound_ref[c]

        def drain_chunk(cnt, sem):

            def drain(k, carry):
                pltpu.make_async_copy(
                    table_ref.at[pl.ds(0, 1), :],
                    gat_ref.at[pl.ds(0, 1), :],
                    sem,
                ).wait()
                return carry

            lax.fori_loop(0, cnt, drain, 0)

        def masked_chunk(c):
            base = c * chunk
            return jnp.where(
                mask_ref[pl.ds(base, chunk), :] != 0.0,
                gat_ref[pl.ds(base, chunk), :],
                0.0,
            )

        cnts = []
        for j in range(N_PEER):
            cnts.append(issue_chunk((my + OFFS[j]) % N_DEV, gsems.at[j]))
        cnt_own = issue_chunk(my, gsems.at[N_PEER])

        p1 = []
        for j in range(N_PEER):
            dst = (my + OFFS[j]) % N_DEV
            c = dst
            drain_chunk(cnts[j], gsems.at[j])
            sbuf_ref[j, :, :] = masked_chunk(c).astype(sbuf_ref.dtype)
            rdma = pltpu.make_async_remote_copy(
                src_ref=sbuf_ref.at[j],
                dst_ref=ra_ref.at[j],
                send_sem=sa_sems.at[j],
                recv_sem=ra_sems.at[j],
                device_id=(dst,),
                device_id_type=pl.DeviceIdType.MESH,
            )
            rdma.start()
            p1.append(rdma)

        drain_chunk(cnt_own, gsems.at[N_PEER])
        mybase = my * chunk
        out_ref[pl.ds(mybase, chunk), :] = masked_chunk(my).astype(out_ref.dtype)

        for j in range(N_PEER):
            p1[j].wait_recv()
            out_ref[pl.ds(mybase, chunk), :] = (
                out_ref[pl.ds(mybase, chunk), :] + ra_ref[j, :, :]
            )

        p2 = []
        for j in range(N_PEER):
            dst = (my + OFFS[j]) % N_DEV
            rdma = pltpu.make_async_remote_copy(
                src_ref=out_ref.at[pl.ds(mybase, chunk), :],
                dst_ref=rb_ref.at[j],
                send_sem=sb_sems.at[j],
                recv_sem=rb_sems.at[j],
                device_id=(dst,),
                device_id_type=pl.DeviceIdType.MESH,
            )
            rdma.start()
            p2.append(rdma)

        for j in range(N_PEER):
            p2[j].wait_recv()
            src = (my - OFFS[j]) % N_DEV
            out_ref[pl.ds(src * chunk, chunk), :] = rb_ref[j, :, :]

        for rdma in p1:
            rdma.wait_send()
        for rdma in p2:
            rdma.wait_send()

    return pl.pallas_call(
        body,
        out_shape=jax.ShapeDtypeStruct((n, d), jnp.bfloat16),
        in_specs=[
            pl.BlockSpec(memory_space=pl.ANY),
            pl.BlockSpec(memory_space=pltpu.SMEM),
            pl.BlockSpec(memory_space=pltpu.SMEM),
            pl.BlockSpec(memory_space=pltpu.VMEM),
        ],
        out_specs=pl.BlockSpec(memory_space=pltpu.VMEM),
        scratch_shapes=[
            pltpu.VMEM((n, d), jnp.float32),
            pltpu.SemaphoreType.DMA((N_DEV,)),
            pltpu.VMEM((N_PEER, chunk, d), jnp.bfloat16),
            pltpu.VMEM((N_PEER, chunk, d), jnp.bfloat16),
            pltpu.VMEM((N_PEER, chunk, d), jnp.bfloat16),
            pltpu.SemaphoreType.DMA((N_PEER,)),
            pltpu.SemaphoreType.DMA((N_PEER,)),
            pltpu.SemaphoreType.DMA((N_PEER,)),
            pltpu.SemaphoreType.DMA((N_PEER,)),
        ],
        compiler_params=pltpu.CompilerParams(collective_id=0),
    )(table, *gather_lists, mask)


def kernel(table, idx):
    v_per = table.shape[0]
    n = idx.shape[0]
    chunk = n // N_DEV
    my = lax.axis_index("i")
    local = idx.astype(jnp.int32) - my * v_per
    mask = (local >= 0) & (local < v_per)

    pos = jnp.arange(n, dtype=jnp.int32)
    pack = jnp.where(mask, jnp.left_shift(local, 11) | pos, 0)
    key = (~mask).astype(jnp.int32)
    _, packs = jax.lax.sort((key, pack), num_keys=1)
    cnts = jnp.sum(mask.reshape(N_DEV, chunk), axis=1).astype(jnp.int32)
    bounds = jnp.concatenate(
        [jnp.zeros((1,), jnp.int32), jnp.cumsum(cnts).astype(jnp.int32)]
    )

    maskf = mask.astype(jnp.float32)[:, None]
    return _fused(table, (packs, bounds), maskf)


# device time: 63827 ns/iter; 1.4419x vs baseline; 1.1665x over previous
import jax
import jax.numpy as jnp
from jax import lax
from jax.experimental import pallas as pl
from jax.experimental.pallas import tpu as pltpu

N_DEV = 4
N_PEER = N_DEV - 1
S = 4
OFFS = (2, 1, 3)


def _fused(table, gather_lists, mask):
    v_per, d = table.shape
    n = mask.shape[0]
    chunk = n // N_DEV
    srows = chunk // S

    def body(table_ref, pack_ref, bound_ref, mask_ref, out_ref,
             gat_ref, gsems, sbuf_ref, ra_ref, rb_ref,
             sa_sems, ra_sems, sb_sems, rb_sems):
        my = lax.axis_index("i")
        mybase = my * chunk

        barrier_sem = pltpu.get_barrier_semaphore()
        for j in range(N_PEER):
            pl.semaphore_signal(
                barrier_sem, inc=1,
                device_id=((my + j + 1) % N_DEV,),
                device_id_type=pl.DeviceIdType.MESH,
            )
        pl.semaphore_wait(barrier_sem, N_PEER)

        def issue_slice(c, s, sem):
            g = c * S + s

            def issue(k, carry):
                p = pack_ref[k]
                r = lax.shift_right_logical(p, 11)
                i = lax.bitwise_and(p, n - 1)
                pltpu.make_async_copy(
                    table_ref.at[pl.ds(r, 1), :],
                    gat_ref.at[pl.ds(i, 1), :],
                    sem,
                ).start()
                return carry

            lax.fori_loop(bound_ref[g], bound_ref[g + 1], issue, 0)
            return bound_ref[g + 1] - bound_ref[g]

        def drain_slice(cnt, sem):

            def drain(k, carry):
                pltpu.make_async_copy(
                    table_ref.at[pl.ds(0, 1), :],
                    gat_ref.at[pl.ds(0, 1), :],
                    sem,
                ).wait()
                return carry

            lax.fori_loop(0, cnt, drain, 0)

        def masked_slice(c, s):
            base = c * chunk + s * srows
            return jnp.where(
                mask_ref[pl.ds(base, srows), :] != 0.0,
                gat_ref[pl.ds(base, srows), :],
                0.0,
            )

        p1 = [[None] * S for _ in range(N_PEER)]
        p2 = [[None] * S for _ in range(N_PEER)]

        def reduce_and_broadcast(s):
            obase = mybase + s * srows
            for j in range(N_PEER):
                p1[j][s].wait_recv()
                out_ref[pl.ds(obase, srows), :] = (
                    out_ref[pl.ds(obase, srows), :]
                    + ra_ref[j, pl.ds(s * srows, srows), :]
                )
            for j in range(N_PEER):
                dst = (my + OFFS[j]) % N_DEV
                rdma = pltpu.make_async_remote_copy(
                    src_ref=out_ref.at[pl.ds(obase, srows), :],
                    dst_ref=rb_ref.at[j, pl.ds(s * srows, srows), :],
                    send_sem=sb_sems.at[j, s],
                    recv_sem=rb_sems.at[j, s],
                    device_id=(dst,),
                    device_id_type=pl.DeviceIdType.MESH,
                )
                rdma.start()
                p2[j][s] = rdma

        for s in range(S):
            cnts = []
            for j in range(N_PEER):
                cnts.append(
                    issue_slice((my + OFFS[j]) % N_DEV, s, gsems.at[j, s])
                )
            cnt_own = issue_slice(my, s, gsems.at[N_PEER, s])

            for j in range(N_PEER):
                c = (my + OFFS[j]) % N_DEV
                drain_slice(cnts[j], gsems.at[j, s])
                sbuf_ref[j, pl.ds(s * srows, srows), :] = (
                    masked_slice(c, s).astype(sbuf_ref.dtype)
                )
                rdma = pltpu.make_async_remote_copy(
                    src_ref=sbuf_ref.at[j, pl.ds(s * srows, srows), :],
                    dst_ref=ra_ref.at[j, pl.ds(s * srows, srows), :],
                    send_sem=sa_sems.at[j, s],
                    recv_sem=ra_sems.at[j, s],
                    device_id=((my + OFFS[j]) % N_DEV,),
                    device_id_type=pl.DeviceIdType.MESH,
                )
                rdma.start()
                p1[j][s] = rdma

            drain_slice(cnt_own, gsems.at[N_PEER, s])
            out_ref[pl.ds(mybase + s * srows, srows), :] = (
                masked_slice(my, s).astype(out_ref.dtype)
            )

            if s >= 1:
                reduce_and_broadcast(s - 1)

        reduce_and_broadcast(S - 1)

        for j in range(N_PEER):
            src = (my - OFFS[j]) % N_DEV
            for s in range(S):
                p2[j][s].wait_recv()
                out_ref[pl.ds(src * chunk + s * srows, srows), :] = (
                    rb_ref[j, pl.ds(s * srows, srows), :]
                )

        for j in range(N_PEER):
            for s in range(S):
                p1[j][s].wait_send()
                p2[j][s].wait_send()

    return pl.pallas_call(
        body,
        out_shape=jax.ShapeDtypeStruct((n, d), jnp.bfloat16),
        in_specs=[
            pl.BlockSpec(memory_space=pl.ANY),
            pl.BlockSpec(memory_space=pltpu.SMEM),
            pl.BlockSpec(memory_space=pltpu.SMEM),
            pl.BlockSpec(memory_space=pltpu.VMEM),
        ],
        out_specs=pl.BlockSpec(memory_space=pltpu.VMEM),
        scratch_shapes=[
            pltpu.VMEM((n, d), jnp.float32),
            pltpu.SemaphoreType.DMA((N_DEV, S)),
            pltpu.VMEM((N_PEER, chunk, d), jnp.bfloat16),
            pltpu.VMEM((N_PEER, chunk, d), jnp.bfloat16),
            pltpu.VMEM((N_PEER, chunk, d), jnp.bfloat16),
            pltpu.SemaphoreType.DMA((N_PEER, S)),
            pltpu.SemaphoreType.DMA((N_PEER, S)),
            pltpu.SemaphoreType.DMA((N_PEER, S)),
            pltpu.SemaphoreType.DMA((N_PEER, S)),
        ],
        compiler_params=pltpu.CompilerParams(collective_id=0),
    )(table, *gather_lists, mask)


def kernel(table, idx):
    v_per = table.shape[0]
    n = idx.shape[0]
    srows = n // (N_DEV * S)
    my = lax.axis_index("i")
    local = idx.astype(jnp.int32) - my * v_per
    mask = (local >= 0) & (local < v_per)

    pos = jnp.arange(n, dtype=jnp.int32)
    pack = jnp.where(mask, jnp.left_shift(local, 11) | pos, 0)
    key = (~mask).astype(jnp.int32)
    _, packs = jax.lax.sort((key, pack), num_keys=1)
    cnts = jnp.sum(mask.reshape(N_DEV * S, srows), axis=1).astype(jnp.int32)
    bounds = jnp.concatenate(
        [jnp.zeros((1,), jnp.int32), jnp.cumsum(cnts).astype(jnp.int32)]
    )

    maskf = mask.astype(jnp.float32)[:, None]
    return _fused(table, (packs, bounds), maskf)


# device time: 62826 ns/iter; 1.4649x vs baseline; 1.0159x over previous
import jax
import jax.numpy as jnp
from jax import lax
from jax.experimental import pallas as pl
from jax.experimental.pallas import tpu as pltpu

N_DEV = 4
N_PEER = N_DEV - 1
S = 4
OFFS = (2, 1, 3)


def _fused(table, gather_lists, mask):
    v_per, d = table.shape
    n = mask.shape[0]
    chunk = n // N_DEV
    srows = chunk // S

    def body(table_ref, pack_ref, bound_ref, mask_ref, out_ref,
             gat_ref, gsems, sbuf_ref, ra_ref, rb_ref,
             sa_sems, ra_sems, sb_sems, rb_sems):
        my = lax.axis_index("i")
        mybase = my * chunk

        barrier_sem = pltpu.get_barrier_semaphore()
        for j in range(N_PEER):
            pl.semaphore_signal(
                barrier_sem, inc=1,
                device_id=((my + j + 1) % N_DEV,),
                device_id_type=pl.DeviceIdType.MESH,
            )
        pl.semaphore_wait(barrier_sem, N_PEER)

        def issue_slice(c, s, sem):
            g = c * S + s

            def issue(k, carry):
                p = pack_ref[k]
                r = lax.shift_right_logical(p, 11)
                i = lax.bitwise_and(p, n - 1)
                pltpu.make_async_copy(
                    table_ref.at[pl.ds(r, 1), :],
                    gat_ref.at[pl.ds(i, 1), :],
                    sem,
                ).start()
                return carry

            lax.fori_loop(bound_ref[g], bound_ref[g + 1], issue, 0)
            return bound_ref[g + 1] - bound_ref[g]

        def drain_slice(cnt, sem):

            def drain(k, carry):
                pltpu.make_async_copy(
                    table_ref.at[pl.ds(0, 1), :],
                    gat_ref.at[pl.ds(0, 1), :],
                    sem,
                ).wait()
                return carry

            lax.fori_loop(0, cnt, drain, 0)

        def masked_slice(c, s):
            base = c * chunk + s * srows
            return jnp.where(
                mask_ref[pl.ds(base, srows), :] != 0.0,
                gat_ref[pl.ds(base, srows), :],
                0.0,
            )

        p1 = [[None] * S for _ in range(N_PEER)]
        p2 = [[None] * S for _ in range(N_PEER)]

        def reduce_and_broadcast(s):
            obase = mybase + s * srows
            for j in range(N_PEER):
                p1[j][s].wait_recv()
                out_ref[pl.ds(obase, srows), :] = (
                    out_ref[pl.ds(obase, srows), :]
                    + ra_ref[j, pl.ds(s * srows, srows), :]
                )
            for j in range(N_PEER):
                dst = (my + OFFS[j]) % N_DEV
                rdma = pltpu.make_async_remote_copy(
                    src_ref=out_ref.at[pl.ds(obase, srows), :],
                    dst_ref=rb_ref.at[j, pl.ds(s * srows, srows), :],
                    send_sem=sb_sems.at[j, s],
                    recv_sem=rb_sems.at[j, s],
                    device_id=(dst,),
                    device_id_type=pl.DeviceIdType.MESH,
                )
                rdma.start()
                p2[j][s] = rdma

        def store_inbound(s):
            for j in range(N_PEER):
                src = (my - OFFS[j]) % N_DEV
                p2[j][s].wait_recv()
                out_ref[pl.ds(src * chunk + s * srows, srows), :] = (
                    rb_ref[j, pl.ds(s * srows, srows), :]
                )

        for s in range(S):
            cnts = []
            for j in range(N_PEER):
                cnts.append(
                    issue_slice((my + OFFS[j]) % N_DEV, s, gsems.at[j, s])
                )
            cnt_own = issue_slice(my, s, gsems.at[N_PEER, s])

            for j in range(N_PEER):
                c = (my + OFFS[j]) % N_DEV
                drain_slice(cnts[j], gsems.at[j, s])
                sbuf_ref[j, pl.ds(s * srows, srows), :] = (
                    masked_slice(c, s).astype(sbuf_ref.dtype)
                )
                rdma = pltpu.make_async_remote_copy(
                    src_ref=sbuf_ref.at[j, pl.ds(s * srows, srows), :],
                    dst_ref=ra_ref.at[j, pl.ds(s * srows, srows), :],
                    send_sem=sa_sems.at[j, s],
                    recv_sem=ra_sems.at[j, s],
                    device_id=((my + OFFS[j]) % N_DEV,),
                    device_id_type=pl.DeviceIdType.MESH,
                )
                rdma.start()
                p1[j][s] = rdma

            drain_slice(cnt_own, gsems.at[N_PEER, s])
            out_ref[pl.ds(mybase + s * srows, srows), :] = (
                masked_slice(my, s).astype(out_ref.dtype)
            )

            if s >= 1:
                reduce_and_broadcast(s - 1)
            if s >= 2:
                store_inbound(s - 2)

        reduce_and_broadcast(S - 1)
        store_inbound(S - 2)
        store_inbound(S - 1)

        for j in range(N_PEER):
            for s in range(S):
                p1[j][s].wait_send()
                p2[j][s].wait_send()

    return pl.pallas_call(
        body,
        out_shape=jax.ShapeDtypeStruct((n, d), jnp.bfloat16),
        in_specs=[
            pl.BlockSpec(memory_space=pl.ANY),
            pl.BlockSpec(memory_space=pltpu.SMEM),
            pl.BlockSpec(memory_space=pltpu.SMEM),
            pl.BlockSpec(memory_space=pltpu.VMEM),
        ],
        out_specs=pl.BlockSpec(memory_space=pltpu.VMEM),
        scratch_shapes=[
            pltpu.VMEM((n, d), jnp.float32),
            pltpu.SemaphoreType.DMA((N_DEV, S)),
            pltpu.VMEM((N_PEER, chunk, d), jnp.bfloat16),
            pltpu.VMEM((N_PEER, chunk, d), jnp.bfloat16),
            pltpu.VMEM((N_PEER, chunk, d), jnp.bfloat16),
            pltpu.SemaphoreType.DMA((N_PEER, S)),
            pltpu.SemaphoreType.DMA((N_PEER, S)),
            pltpu.SemaphoreType.DMA((N_PEER, S)),
            pltpu.SemaphoreType.DMA((N_PEER, S)),
        ],
        compiler_params=pltpu.CompilerParams(collective_id=0),
    )(table, *gather_lists, mask)


def kernel(table, idx):
    v_per = table.shape[0]
    n = idx.shape[0]
    srows = n // (N_DEV * S)
    my = lax.axis_index("i")
    local = idx.astype(jnp.int32) - my * v_per
    mask = (local >= 0) & (local < v_per)

    pos = jnp.arange(n, dtype=jnp.int32)
    pack = jnp.where(mask, jnp.left_shift(local, 11) | pos, 0)
    key = (~mask).astype(jnp.int32)
    _, packs = jax.lax.sort((key, pack), num_keys=1)
    cnts = jnp.sum(mask.reshape(N_DEV * S, srows), axis=1).astype(jnp.int32)
    bounds = jnp.concatenate(
        [jnp.zeros((1,), jnp.int32), jnp.cumsum(cnts).astype(jnp.int32)]
    )

    maskf = mask.astype(jnp.float32)[:, None]
    return _fused(table, (packs, bounds), maskf)
